# baseline (speedup 1.0000x reference)
import jax
import jax.numpy as jnp
from jax import lax
from jax.experimental import pallas as pl
from jax.experimental.pallas import tpu as pltpu
from jax.experimental.pallas import tpu_sc as plsc

_B = 128
_N = 32768
_L = 16
_NC = 2
_NS = 16
_NW = _NC * _NS
_RPW = _B // _NW
_NBUF = 3
_TCROWS = 16


def _argmax_body(
    in_hbm, idx_hbm, inbuf0, inbuf1, inbuf2, idxv,
    sem_a, sem_b, sem_c, sem_o,
):
    inbufs = (inbuf0, inbuf1, inbuf2)
    in_sems = (sem_a, sem_b, sem_c)
    cid = lax.axis_index("c")
    sid = lax.axis_index("s")
    wid = sid * _NC + cid
    base_row = wid * _RPW

    in_handles = [
        pltpu.async_copy(in_hbm.at[base_row + r], inbufs[r % _NBUF], in_sems[r % _NBUF])
        for r in range(_NBUF)
    ]

    lane = lax.iota(jnp.int32, _L)
    neg_inf = jnp.full((_L,), -jnp.inf, jnp.float32)

    def make_argmax_body(buf):
        def argmax_body(i, carry):
            v0, i0, v1, i1, cur = carry
            x0 = buf[pl.ds(i * (2 * _L), _L)]
            x1 = buf[pl.ds(i * (2 * _L) + _L, _L)]
            b0 = x0 > v0
            b1 = x1 > v1
            v0 = jnp.where(b0, x0, v0)
            i0 = jnp.where(b0, cur, i0)
            v1 = jnp.where(b1, x1, v1)
            i1 = jnp.where(b1, cur + _L, i1)
            return v0, i0, v1, i1, cur + 2 * _L

        return argmax_body

    winners = jnp.zeros((_L,), jnp.int32)
    for r in range(_RPW):
        in_handles[r].wait()
        v0, i0, v1, i1, _ = lax.fori_loop(
            0,
            _N // (2 * _L),
            make_argmax_body(inbufs[r % _NBUF]),
            (neg_inf, lane, neg_inf, lane + _L, lane),
            unroll=4,
        )
        if r + _NBUF < _RPW:
            in_handles.append(
                pltpu.async_copy(
                    in_hbm.at[base_row + r + _NBUF],
                    inbufs[r % _NBUF],
                    in_sems[r % _NBUF],
                )
            )
        take1 = jnp.logical_or(v1 > v0, jnp.logical_and(v1 == v0, i1 < i0))
        best_v = jnp.where(take1, v1, v0)
        best_i = jnp.where(take1, i1, i0)
        for s in (8, 4, 2, 1):
            perm = jnp.bitwise_xor(lane, s)
            ov = best_v.at[perm].get(mode="promise_in_bounds")
            oi = best_i.at[perm].get(mode="promise_in_bounds")
            better = jnp.logical_or(
                ov > best_v, jnp.logical_and(ov == best_v, oi < best_i)
            )
            best_v = jnp.where(better, ov, best_v)
            best_i = jnp.where(better, oi, best_i)
        winners = jnp.where(lane == r, best_i[0], winners)

    idxv[:] = winners
    pltpu.async_copy(idxv, idx_hbm.at[wid], sem_o).wait()


def _sc_argmax(tensor):
    mesh = plsc.VectorSubcoreMesh(
        core_axis_name="c", subcore_axis_name="s", num_cores=_NC, num_subcores=_NS
    )
    f = pl.kernel(
        _argmax_body,
        out_type=jax.ShapeDtypeStruct((_NW, _L), jnp.int32),
        mesh=mesh,
        scratch_types=[
            pltpu.VMEM((_N,), jnp.float32),
            pltpu.VMEM((_N,), jnp.float32),
            pltpu.VMEM((_N,), jnp.float32),
            pltpu.VMEM((_L,), jnp.int32),
            pltpu.SemaphoreType.DMA,
            pltpu.SemaphoreType.DMA,
            pltpu.SemaphoreType.DMA,
            pltpu.SemaphoreType.DMA,
        ],
    )
    return f(tensor)


def _onehot_block(idx_ref, out_ref):
    g = pl.program_id(0)
    col = lax.broadcasted_iota(jnp.int32, (_TCROWS, _N), 1)
    tgt = jnp.stack([idx_ref[g * _TCROWS + r] for r in range(_TCROWS)])
    out_ref[...] = (col == tgt[:, None]).astype(jnp.float32)


def _tc_onehot(idx):
    return pl.pallas_call(
        _onehot_block,
        grid=(_B // _TCROWS,),
        in_specs=[pl.BlockSpec(memory_space=pltpu.SMEM)],
        out_specs=pl.BlockSpec((_TCROWS, _N), lambda i: (i, 0)),
        out_shape=jax.ShapeDtypeStruct((_B, _N), jnp.float32),
    )(idx)


def kernel(tensor):
    idx2d = _sc_argmax(tensor)
    idx = idx2d[:, :_RPW].reshape(_B)
    return _tc_onehot(idx)


# speedup vs baseline: 1.0669x; 1.0669x over previous
import jax
import jax.numpy as jnp
from jax import lax
from jax.experimental import pallas as pl
from jax.experimental.pallas import tpu as pltpu
from jax.experimental.pallas import tpu_sc as plsc

_B = 128
_N = 32768
_L = 16
_NC = 2
_NS = 16
_NW = _NC * _NS
_RPW = _B // _NW
_QPR = 4
_Q = _N // _QPR
_NBUF = 8
_NQ = _RPW * _QPR
_TCROWS = 16


def _argmax_body(in_hbm, idx_hbm, *refs):
    inbufs = refs[:_NBUF]
    idxv = refs[_NBUF]
    in_sems = refs[_NBUF + 1 : _NBUF + 1 + _NBUF]
    sem_o = refs[_NBUF + 1 + _NBUF]

    cid = lax.axis_index("c")
    sid = lax.axis_index("s")
    wid = sid * _NC + cid
    base_row = wid * _RPW

    def chunk_src(q):
        r, p = divmod(q, _QPR)
        return in_hbm.at[base_row + r, pl.ds(p * _Q, _Q)]

    in_handles = [
        pltpu.async_copy(chunk_src(q), inbufs[q % _NBUF], in_sems[q % _NBUF])
        for q in range(_NBUF)
    ]

    lane = lax.iota(jnp.int32, _L)
    neg_inf = jnp.full((_L,), -jnp.inf, jnp.float32)

    def make_argmax_body(buf, base):
        def argmax_body(i, carry):
            v0, i0, v1, i1 = carry
            cur = base + i * (2 * _L) + lane
            x0 = buf[pl.ds(i * (2 * _L), _L)]
            x1 = buf[pl.ds(i * (2 * _L) + _L, _L)]
            b0 = x0 > v0
            b1 = x1 > v1
            v0 = jnp.where(b0, x0, v0)
            i0 = jnp.where(b0, cur, i0)
            v1 = jnp.where(b1, x1, v1)
            i1 = jnp.where(b1, cur + _L, i1)
            return v0, i0, v1, i1

        return argmax_body

    winners = jnp.zeros((_L,), jnp.int32)
    for r in range(_RPW):
        carry = (neg_inf, lane, neg_inf, lane + _L)
        for p in range(_QPR):
            q = r * _QPR + p
            in_handles[q].wait()
            carry = lax.fori_loop(
                0,
                _Q // (2 * _L),
                make_argmax_body(inbufs[q % _NBUF], p * _Q),
                carry,
                unroll=4,
            )
            if q + _NBUF < _NQ:
                in_handles.append(
                    pltpu.async_copy(
                        chunk_src(q + _NBUF),
                        inbufs[q % _NBUF],
                        in_sems[q % _NBUF],
                    )
                )
        v0, i0, v1, i1 = carry
        take1 = jnp.logical_or(v1 > v0, jnp.logical_and(v1 == v0, i1 < i0))
        best_v = jnp.where(take1, v1, v0)
        best_i = jnp.where(take1, i1, i0)
        for s in (8, 4, 2, 1):
            perm = jnp.bitwise_xor(lane, s)
            ov = best_v.at[perm].get(mode="promise_in_bounds")
            oi = best_i.at[perm].get(mode="promise_in_bounds")
            better = jnp.logical_or(
                ov > best_v, jnp.logical_and(ov == best_v, oi < best_i)
            )
            best_v = jnp.where(better, ov, best_v)
            best_i = jnp.where(better, oi, best_i)
        winners = jnp.where(lane == r, best_i[0], winners)

    idxv[:] = winners
    pltpu.async_copy(idxv, idx_hbm.at[wid], sem_o).wait()


def _sc_argmax(tensor):
    mesh = plsc.VectorSubcoreMesh(
        core_axis_name="c", subcore_axis_name="s", num_cores=_NC, num_subcores=_NS
    )
    f = pl.kernel(
        _argmax_body,
        out_type=jax.ShapeDtypeStruct((_NW, _L), jnp.int32),
        mesh=mesh,
        scratch_types=(
            [pltpu.VMEM((_Q,), jnp.float32) for _ in range(_NBUF)]
            + [pltpu.VMEM((_L,), jnp.int32)]
            + [pltpu.SemaphoreType.DMA for _ in range(_NBUF + 1)]
        ),
    )
    return f(tensor)


def _onehot_block(idx_ref, out_ref):
    g = pl.program_id(0)
    col = lax.broadcasted_iota(jnp.int32, (_TCROWS, _N), 1)
    tgt = jnp.stack(
        [idx_ref[g * (_TCROWS // _RPW) + k // _RPW, k % _RPW] for k in range(_TCROWS)]
    )
    out_ref[...] = (col == tgt[:, None]).astype(jnp.float32)


def _tc_onehot(idx2d):
    return pl.pallas_call(
        _onehot_block,
        grid=(_B // _TCROWS,),
        in_specs=[pl.BlockSpec(memory_space=pltpu.SMEM)],
        out_specs=pl.BlockSpec((_TCROWS, _N), lambda i: (i, 0)),
        out_shape=jax.ShapeDtypeStruct((_B, _N), jnp.float32),
    )(idx2d)


def kernel(tensor):
    idx2d = _sc_argmax(tensor)
    return _tc_onehot(idx2d)


# speedup vs baseline: 1.1631x; 1.0901x over previous
import jax
import jax.numpy as jnp
from jax import lax
from jax.experimental import pallas as pl
from jax.experimental.pallas import tpu as pltpu
from jax.experimental.pallas import tpu_sc as plsc

_B = 128
_N = 32768
_L = 16
_NC = 2
_NS = 16
_NW = _NC * _NS
_SCROWS = 64
_RPW = _SCROWS // _NW
_QPR = 4
_Q = _N // _QPR
_NBUF = 8
_NQ = _RPW * _QPR
_TCROWS = 16


def _argmax_body(in_hbm, idx_hbm, *refs):
    inbufs = refs[:_NBUF]
    idxv = refs[_NBUF]
    in_sems = refs[_NBUF + 1 : _NBUF + 1 + _NBUF]
    sem_o = refs[_NBUF + 1 + _NBUF]

    cid = lax.axis_index("c")
    sid = lax.axis_index("s")
    wid = sid * _NC + cid
    base_row = (_B - _SCROWS) + wid * _RPW

    def chunk_src(q):
        r, p = divmod(q, _QPR)
        return in_hbm.at[base_row + r, pl.ds(p * _Q, _Q)]

    in_handles = [
        pltpu.async_copy(chunk_src(q), inbufs[q % _NBUF], in_sems[q % _NBUF])
        for q in range(min(_NBUF, _NQ))
    ]

    lane = lax.iota(jnp.int32, _L)
    neg_inf = jnp.full((_L,), -jnp.inf, jnp.float32)

    def make_argmax_body(buf, base):
        def argmax_body(i, carry):
            v0, i0, v1, i1 = carry
            cur = base + i * (2 * _L) + lane
            x0 = buf[pl.ds(i * (2 * _L), _L)]
            x1 = buf[pl.ds(i * (2 * _L) + _L, _L)]
            b0 = x0 > v0
            b1 = x1 > v1
            v0 = jnp.where(b0, x0, v0)
            i0 = jnp.where(b0, cur, i0)
            v1 = jnp.where(b1, x1, v1)
            i1 = jnp.where(b1, cur + _L, i1)
            return v0, i0, v1, i1

        return argmax_body

    winners = jnp.zeros((_L,), jnp.int32)
    for r in range(_RPW):
        carry = (neg_inf, lane, neg_inf, lane + _L)
        for p in range(_QPR):
            q = r * _QPR + p
            in_handles[q].wait()
            carry = lax.fori_loop(
                0,
                _Q // (2 * _L),
                make_argmax_body(inbufs[q % _NBUF], p * _Q),
                carry,
                unroll=4,
            )
            if q + _NBUF < _NQ:
                in_handles.append(
                    pltpu.async_copy(
                        chunk_src(q + _NBUF),
                        inbufs[q % _NBUF],
                        in_sems[q % _NBUF],
                    )
                )
        v0, i0, v1, i1 = carry
        take1 = jnp.logical_or(v1 > v0, jnp.logical_and(v1 == v0, i1 < i0))
        best_v = jnp.where(take1, v1, v0)
        best_i = jnp.where(take1, i1, i0)
        for s in (8, 4, 2, 1):
            perm = jnp.bitwise_xor(lane, s)
            ov = best_v.at[perm].get(mode="promise_in_bounds")
            oi = best_i.at[perm].get(mode="promise_in_bounds")
            better = jnp.logical_or(
                ov > best_v, jnp.logical_and(ov == best_v, oi < best_i)
            )
            best_v = jnp.where(better, ov, best_v)
            best_i = jnp.where(better, oi, best_i)
        winners = jnp.where(lane == r, best_i[0], winners)

    idxv[:] = winners
    pltpu.async_copy(idxv, idx_hbm.at[wid], sem_o).wait()


def _sc_argmax(tensor):
    mesh = plsc.VectorSubcoreMesh(
        core_axis_name="c", subcore_axis_name="s", num_cores=_NC, num_subcores=_NS
    )
    f = pl.kernel(
        _argmax_body,
        out_type=jax.ShapeDtypeStruct((_NW, _L), jnp.int32),
        mesh=mesh,
        scratch_types=(
            [pltpu.VMEM((_Q,), jnp.float32) for _ in range(_NBUF)]
            + [pltpu.VMEM((_L,), jnp.int32)]
            + [pltpu.SemaphoreType.DMA for _ in range(_NBUF + 1)]
        ),
    )
    return f(tensor)


def _tc_argmax_block(in_ref, out_ref):
    x = in_ref[...]
    col = lax.broadcasted_iota(jnp.int32, (_TCROWS, _N), 1)
    m = jnp.max(x, axis=1, keepdims=True)
    idx = jnp.min(jnp.where(x == m, col, _N), axis=1)
    out_ref[...] = idx.reshape(1, 1, _TCROWS)


def _tc_argmax(tensor):
    nblk = (_B - _SCROWS) // _TCROWS
    return pl.pallas_call(
        _tc_argmax_block,
        grid=(nblk,),
        in_specs=[pl.BlockSpec((_TCROWS, _N), lambda i: (i, 0))],
        out_specs=pl.BlockSpec((1, 1, _TCROWS), lambda i: (i, 0, 0)),
        out_shape=jax.ShapeDtypeStruct((nblk, 1, _TCROWS), jnp.int32),
    )(tensor)


def _onehot_block(idxtc_ref, idxsc_ref, out_ref):
    g = pl.program_id(0)
    gtc = jnp.minimum(g, (_B - _SCROWS) // _TCROWS - 1)
    col = lax.broadcasted_iota(jnp.int32, (_TCROWS, _N), 1)
    tgts = []
    for k in range(_TCROWS):
        t_tc = idxtc_ref[gtc, 0, k]
        sc_row = g * _TCROWS + k - (_B - _SCROWS)
        sc_row = jnp.clip(sc_row, 0, _SCROWS - 1)
        t_sc = idxsc_ref[sc_row // _RPW, sc_row % _RPW]
        tgts.append(jnp.where(g * _TCROWS < _B - _SCROWS, t_tc, t_sc))
    tgt = jnp.stack(tgts)
    out_ref[...] = (col == tgt[:, None]).astype(jnp.float32)


def _tc_onehot(idx_tc, idx_sc):
    return pl.pallas_call(
        _onehot_block,
        grid=(_B // _TCROWS,),
        in_specs=[
            pl.BlockSpec(memory_space=pltpu.SMEM),
            pl.BlockSpec(memory_space=pltpu.SMEM),
        ],
        out_specs=pl.BlockSpec((_TCROWS, _N), lambda i: (i, 0)),
        out_shape=jax.ShapeDtypeStruct((_B, _N), jnp.float32),
    )(idx_tc, idx_sc)


def kernel(tensor):
    idx_sc = _sc_argmax(tensor)
    idx_tc = _tc_argmax(tensor)
    return _tc_onehot(idx_tc, idx_sc)


# speedup vs baseline: 1.2360x; 1.0627x over previous
import jax
import jax.numpy as jnp
from jax import lax
from jax.experimental import pallas as pl
from jax.experimental.pallas import tpu as pltpu
from jax.experimental.pallas import tpu_sc as plsc

_B = 128
_N = 32768
_L = 16
_NC = 2
_NS = 16
_NW = _NC * _NS
_SCROWS = 32
_RPW = _SCROWS // _NW
_QPR = 4
_Q = _N // _QPR
_NBUF = 8
_NQ = _RPW * _QPR
_TCROWS = 32


def _argmax_body(in_hbm, idx_hbm, *refs):
    inbufs = refs[:_NBUF]
    idxv = refs[_NBUF]
    in_sems = refs[_NBUF + 1 : _NBUF + 1 + _NBUF]
    sem_o = refs[_NBUF + 1 + _NBUF]

    cid = lax.axis_index("c")
    sid = lax.axis_index("s")
    wid = sid * _NC + cid
    base_row = (_B - _SCROWS) + wid * _RPW

    def chunk_src(q):
        r, p = divmod(q, _QPR)
        return in_hbm.at[base_row + r, pl.ds(p * _Q, _Q)]

    in_handles = [
        pltpu.async_copy(chunk_src(q), inbufs[q % _NBUF], in_sems[q % _NBUF])
        for q in range(min(_NBUF, _NQ))
    ]

    lane = lax.iota(jnp.int32, _L)
    neg_inf = jnp.full((_L,), -jnp.inf, jnp.float32)

    def make_argmax_body(buf, base):
        def argmax_body(i, carry):
            v0, i0, v1, i1 = carry
            cur = base + i * (2 * _L) + lane
            x0 = buf[pl.ds(i * (2 * _L), _L)]
            x1 = buf[pl.ds(i * (2 * _L) + _L, _L)]
            b0 = x0 > v0
            b1 = x1 > v1
            v0 = jnp.where(b0, x0, v0)
            i0 = jnp.where(b0, cur, i0)
            v1 = jnp.where(b1, x1, v1)
            i1 = jnp.where(b1, cur + _L, i1)
            return v0, i0, v1, i1

        return argmax_body

    winners = jnp.zeros((_L,), jnp.int32)
    for r in range(_RPW):
        carry = (neg_inf, lane, neg_inf, lane + _L)
        for p in range(_QPR):
            q = r * _QPR + p
            in_handles[q].wait()
            carry = lax.fori_loop(
                0,
                _Q // (2 * _L),
                make_argmax_body(inbufs[q % _NBUF], p * _Q),
                carry,
                unroll=4,
            )
            if q + _NBUF < _NQ:
                in_handles.append(
                    pltpu.async_copy(
                        chunk_src(q + _NBUF),
                        inbufs[q % _NBUF],
                        in_sems[q % _NBUF],
                    )
                )
        v0, i0, v1, i1 = carry
        take1 = jnp.logical_or(v1 > v0, jnp.logical_and(v1 == v0, i1 < i0))
        best_v = jnp.where(take1, v1, v0)
        best_i = jnp.where(take1, i1, i0)
        for s in (8, 4, 2, 1):
            perm = jnp.bitwise_xor(lane, s)
            ov = best_v.at[perm].get(mode="promise_in_bounds")
            oi = best_i.at[perm].get(mode="promise_in_bounds")
            better = jnp.logical_or(
                ov > best_v, jnp.logical_and(ov == best_v, oi < best_i)
            )
            best_v = jnp.where(better, ov, best_v)
            best_i = jnp.where(better, oi, best_i)
        winners = jnp.where(lane == r, best_i[0], winners)

    idxv[:] = winners
    pltpu.async_copy(idxv, idx_hbm.at[wid], sem_o).wait()


def _sc_argmax(tensor):
    mesh = plsc.VectorSubcoreMesh(
        core_axis_name="c", subcore_axis_name="s", num_cores=_NC, num_subcores=_NS
    )
    f = pl.kernel(
        _argmax_body,
        out_type=jax.ShapeDtypeStruct((_NW, _L), jnp.int32),
        mesh=mesh,
        scratch_types=(
            [pltpu.VMEM((_Q,), jnp.float32) for _ in range(_NBUF)]
            + [pltpu.VMEM((_L,), jnp.int32)]
            + [pltpu.SemaphoreType.DMA for _ in range(_NBUF + 1)]
        ),
    )
    return f(tensor)


def _tc_argmax_block(in_ref, out_ref):
    x = in_ref[...]
    col = lax.broadcasted_iota(jnp.int32, (_TCROWS, _N), 1)
    m = jnp.max(x, axis=1, keepdims=True)
    idx = jnp.min(jnp.where(x == m, col, _N), axis=1)
    out_ref[...] = idx.reshape(1, 1, _TCROWS)


def _tc_argmax(tensor):
    nblk = (_B - _SCROWS) // _TCROWS
    return pl.pallas_call(
        _tc_argmax_block,
        grid=(nblk,),
        in_specs=[pl.BlockSpec((_TCROWS, _N), lambda i: (i, 0))],
        out_specs=pl.BlockSpec((1, 1, _TCROWS), lambda i: (i, 0, 0)),
        out_shape=jax.ShapeDtypeStruct((nblk, 1, _TCROWS), jnp.int32),
    )(tensor)


def _onehot_block(idxtc_ref, idxsc_ref, out_ref):
    g = pl.program_id(0)
    gtc = jnp.minimum(g, (_B - _SCROWS) // _TCROWS - 1)
    col = lax.broadcasted_iota(jnp.int32, (_TCROWS, _N), 1)
    tgts = []
    for k in range(_TCROWS):
        t_tc = idxtc_ref[gtc, 0, k]
        sc_row = g * _TCROWS + k - (_B - _SCROWS)
        sc_row = jnp.clip(sc_row, 0, _SCROWS - 1)
        t_sc = idxsc_ref[sc_row // _RPW, sc_row % _RPW]
        tgts.append(jnp.where(g * _TCROWS < _B - _SCROWS, t_tc, t_sc))
    tgt = jnp.stack(tgts)
    out_ref[...] = (col == tgt[:, None]).astype(jnp.float32)


def _tc_onehot(idx_tc, idx_sc):
    return pl.pallas_call(
        _onehot_block,
        grid=(_B // _TCROWS,),
        in_specs=[
            pl.BlockSpec(memory_space=pltpu.SMEM),
            pl.BlockSpec(memory_space=pltpu.SMEM),
        ],
        out_specs=pl.BlockSpec((_TCROWS, _N), lambda i: (i, 0)),
        out_shape=jax.ShapeDtypeStruct((_B, _N), jnp.float32),
    )(idx_tc, idx_sc)


def kernel(tensor):
    idx_sc = _sc_argmax(tensor)
    idx_tc = _tc_argmax(tensor)
    return _tc_onehot(idx_tc, idx_sc)


# speedup vs baseline: 1.2504x; 1.0116x over previous
import jax
import jax.numpy as jnp
from jax import lax
from jax.experimental import pallas as pl
from jax.experimental.pallas import tpu as pltpu
from jax.experimental.pallas import tpu_sc as plsc

_B = 128
_N = 32768
_L = 16
_NC = 2
_NS = 16
_NW = _NC * _NS
_SCROWS = _NW
_TCR = _B - _SCROWS
_QPR = 4
_Q = _N // _QPR
_TCROWS = 32


def _sc_body(in_hbm, out_hbm, *refs):
    inbufs = refs[:_QPR]
    zbuf = refs[_QPR]
    patch = refs[_QPR + 1]
    in_sems = refs[_QPR + 2 : 2 * _QPR + 2]
    sem_z = refs[2 * _QPR + 2]
    sem_p = refs[2 * _QPR + 3]

    cid = lax.axis_index("c")
    sid = lax.axis_index("s")
    wid = sid * _NC + cid
    row = _TCR + wid

    in_handles = [
        pltpu.async_copy(
            in_hbm.at[row, pl.ds(p * _Q, _Q)], inbufs[p], in_sems[p]
        )
        for p in range(_QPR)
    ]

    zero16 = jnp.zeros((_L,), jnp.float32)

    def zero_body(i, _):
        zbuf[pl.ds(i * _L, _L)] = zero16
        return 0

    lax.fori_loop(0, _Q // _L, zero_body, 0, unroll=8)
    z_handles = [
        pltpu.async_copy(zbuf, out_hbm.at[row, pl.ds(p * _Q, _Q)], sem_z)
        for p in range(_QPR)
    ]

    lane = lax.iota(jnp.int32, _L)
    neg_inf = jnp.full((_L,), -jnp.inf, jnp.float32)

    def make_argmax_body(buf, base):
        def argmax_body(i, carry):
            v0, i0, v1, i1 = carry
            cur = base + i * (2 * _L) + lane
            x0 = buf[pl.ds(i * (2 * _L), _L)]
            x1 = buf[pl.ds(i * (2 * _L) + _L, _L)]
            b0 = x0 > v0
            b1 = x1 > v1
            v0 = jnp.where(b0, x0, v0)
            i0 = jnp.where(b0, cur, i0)
            v1 = jnp.where(b1, x1, v1)
            i1 = jnp.where(b1, cur + _L, i1)
            return v0, i0, v1, i1

        return argmax_body

    carry = (neg_inf, lane, neg_inf, lane + _L)
    for p in range(_QPR):
        in_handles[p].wait()
        carry = lax.fori_loop(
            0, _Q // (2 * _L), make_argmax_body(inbufs[p], p * _Q), carry,
            unroll=4,
        )
    v0, i0, v1, i1 = carry
    take1 = jnp.logical_or(v1 > v0, jnp.logical_and(v1 == v0, i1 < i0))
    best_v = jnp.where(take1, v1, v0)
    best_i = jnp.where(take1, i1, i0)
    for s in (8, 4, 2, 1):
        perm = jnp.bitwise_xor(lane, s)
        ov = best_v.at[perm].get(mode="promise_in_bounds")
        oi = best_i.at[perm].get(mode="promise_in_bounds")
        better = jnp.logical_or(
            ov > best_v, jnp.logical_and(ov == best_v, oi < best_i)
        )
        best_v = jnp.where(better, ov, best_v)
        best_i = jnp.where(better, oi, best_i)
    idx = best_i[0]
    off = jnp.bitwise_and(idx, _L - 1)
    blk = pl.multiple_of(jnp.bitwise_and(idx, -_L), _L)
    patch[:] = jnp.where(lane == off, 1.0, 0.0).astype(jnp.float32)

    for h in z_handles:
        h.wait()
    pltpu.async_copy(
        patch, out_hbm.at[row, pl.ds(blk, _L)], sem_p
    ).wait()


def _sc_partial(tensor):
    mesh = plsc.VectorSubcoreMesh(
        core_axis_name="c", subcore_axis_name="s", num_cores=_NC, num_subcores=_NS
    )
    f = pl.kernel(
        _sc_body,
        out_type=jax.ShapeDtypeStruct((_B, _N), jnp.float32),
        mesh=mesh,
        scratch_types=(
            [pltpu.VMEM((_Q,), jnp.float32) for _ in range(_QPR)]
            + [pltpu.VMEM((_Q,), jnp.float32), pltpu.VMEM((_L,), jnp.float32)]
            + [pltpu.SemaphoreType.DMA for _ in range(_QPR + 2)]
        ),
    )
    return f(tensor)


def _tc_argmax_block(in_ref, out_ref):
    x = in_ref[...]
    col = lax.broadcasted_iota(jnp.int32, (_TCROWS, _N), 1)
    m = jnp.max(x, axis=1, keepdims=True)
    idx = jnp.min(jnp.where(x == m, col, _N), axis=1)
    out_ref[...] = idx.reshape(1, 1, _TCROWS)


def _tc_argmax(tensor):
    nblk = _TCR // _TCROWS
    return pl.pallas_call(
        _tc_argmax_block,
        grid=(nblk,),
        in_specs=[pl.BlockSpec((_TCROWS, _N), lambda i: (i, 0))],
        out_specs=pl.BlockSpec((1, 1, _TCROWS), lambda i: (i, 0, 0)),
        out_shape=jax.ShapeDtypeStruct((nblk, 1, _TCROWS), jnp.int32),
    )(tensor)


def _onehot_block(idx_ref, carry_ref, out_ref):
    del carry_ref
    g = pl.program_id(0)
    col = lax.broadcasted_iota(jnp.int32, (_TCROWS, _N), 1)
    tgt = jnp.stack([idx_ref[g, 0, k] for k in range(_TCROWS)])
    out_ref[...] = (col == tgt[:, None]).astype(jnp.float32)


def _tc_onehot(idx_tc, y_sc):
    return pl.pallas_call(
        _onehot_block,
        grid=(_TCR // _TCROWS,),
        in_specs=[
            pl.BlockSpec(memory_space=pltpu.SMEM),
            pl.BlockSpec(memory_space=pl.ANY),
        ],
        out_specs=pl.BlockSpec((_TCROWS, _N), lambda i: (i, 0)),
        out_shape=jax.ShapeDtypeStruct((_B, _N), jnp.float32),
        input_output_aliases={1: 0},
    )(idx_tc, y_sc)


def kernel(tensor):
    y_sc = _sc_partial(tensor)
    idx_tc = _tc_argmax(tensor)
    return _tc_onehot(idx_tc, y_sc)


# speedup vs baseline: 1.2822x; 1.0254x over previous
import jax
import jax.numpy as jnp
from jax import lax
from jax.experimental import pallas as pl
from jax.experimental.pallas import tpu as pltpu
from jax.experimental.pallas import tpu_sc as plsc

_B = 128
_N = 32768
_L = 16
_NC = 2
_NS = 16
_NW = _NC * _NS
_SCROWS = _NW
_TCR = _B - _SCROWS
_QPR = 8
_Q = _N // _QPR
_TCROWS = 32


def _sc_body(in_hbm, out_hbm, *refs):
    inbufs = refs[:_QPR]
    zbuf = refs[_QPR]
    patch = refs[_QPR + 1]
    in_sems = refs[_QPR + 2 : 2 * _QPR + 2]
    sem_z = refs[2 * _QPR + 2]
    sem_p = refs[2 * _QPR + 3]

    cid = lax.axis_index("c")
    sid = lax.axis_index("s")
    wid = sid * _NC + cid
    row = _TCR + wid

    in_handles = [
        pltpu.async_copy(
            in_hbm.at[row, pl.ds(p * _Q, _Q)], inbufs[p], in_sems[p]
        )
        for p in range(_QPR)
    ]

    zero16 = jnp.zeros((_L,), jnp.float32)

    def zero_body(i, _):
        zbuf[pl.ds(i * _L, _L)] = zero16
        return 0

    lax.fori_loop(0, _Q // _L, zero_body, 0, unroll=8)
    z_handles = [
        pltpu.async_copy(zbuf, out_hbm.at[row, pl.ds(p * _Q, _Q)], sem_z)
        for p in range(_QPR)
    ]

    lane = lax.iota(jnp.int32, _L)
    neg_inf = jnp.full((_L,), -jnp.inf, jnp.float32)

    def make_argmax_body(buf, base):
        def argmax_body(i, carry):
            v0, i0, v1, i1 = carry
            cur = base + i * (2 * _L) + lane
            x0 = buf[pl.ds(i * (2 * _L), _L)]
            x1 = buf[pl.ds(i * (2 * _L) + _L, _L)]
            b0 = x0 > v0
            b1 = x1 > v1
            v0 = jnp.where(b0, x0, v0)
            i0 = jnp.where(b0, cur, i0)
            v1 = jnp.where(b1, x1, v1)
            i1 = jnp.where(b1, cur + _L, i1)
            return v0, i0, v1, i1

        return argmax_body

    carry = (neg_inf, lane, neg_inf, lane + _L)
    for p in range(_QPR):
        in_handles[p].wait()
        carry = lax.fori_loop(
            0, _Q // (2 * _L), make_argmax_body(inbufs[p], p * _Q), carry,
            unroll=4,
        )
    v0, i0, v1, i1 = carry
    take1 = jnp.logical_or(v1 > v0, jnp.logical_and(v1 == v0, i1 < i0))
    best_v = jnp.where(take1, v1, v0)
    best_i = jnp.where(take1, i1, i0)
    for s in (8, 4, 2, 1):
        perm = jnp.bitwise_xor(lane, s)
        ov = best_v.at[perm].get(mode="promise_in_bounds")
        oi = best_i.at[perm].get(mode="promise_in_bounds")
        better = jnp.logical_or(
            ov > best_v, jnp.logical_and(ov == best_v, oi < best_i)
        )
        best_v = jnp.where(better, ov, best_v)
        best_i = jnp.where(better, oi, best_i)
    idx = best_i[0]
    off = jnp.bitwise_and(idx, _L - 1)
    blk = pl.multiple_of(jnp.bitwise_and(idx, -_L), _L)
    patch[:] = jnp.where(lane == off, 1.0, 0.0).astype(jnp.float32)

    for h in z_handles:
        h.wait()
    pltpu.async_copy(
        patch, out_hbm.at[row, pl.ds(blk, _L)], sem_p
    ).wait()


def _sc_partial(tensor):
    mesh = plsc.VectorSubcoreMesh(
        core_axis_name="c", subcore_axis_name="s", num_cores=_NC, num_subcores=_NS
    )
    f = pl.kernel(
        _sc_body,
        out_type=jax.ShapeDtypeStruct((_B, _N), jnp.float32),
        mesh=mesh,
        scratch_types=(
            [pltpu.VMEM((_Q,), jnp.float32) for _ in range(_QPR)]
            + [pltpu.VMEM((_Q,), jnp.float32), pltpu.VMEM((_L,), jnp.float32)]
            + [pltpu.SemaphoreType.DMA for _ in range(_QPR + 2)]
        ),
    )
    return f(tensor)


def _tc_argmax_block(in_ref, out_ref):
    x = in_ref[...]
    col = lax.broadcasted_iota(jnp.int32, (_TCROWS, _N), 1)
    m = jnp.max(x, axis=1, keepdims=True)
    idx = jnp.min(jnp.where(x == m, col, _N), axis=1)
    out_ref[...] = idx.reshape(1, 1, _TCROWS)


def _tc_argmax(tensor):
    nblk = _TCR // _TCROWS
    return pl.pallas_call(
        _tc_argmax_block,
        grid=(nblk,),
        in_specs=[pl.BlockSpec((_TCROWS, _N), lambda i: (i, 0))],
        out_specs=pl.BlockSpec((1, 1, _TCROWS), lambda i: (i, 0, 0)),
        out_shape=jax.ShapeDtypeStruct((nblk, 1, _TCROWS), jnp.int32),
    )(tensor)


def _onehot_block(idx_ref, carry_ref, out_ref):
    del carry_ref
    g = pl.program_id(0)
    col = lax.broadcasted_iota(jnp.int32, (_TCROWS, _N), 1)
    tgt = jnp.stack([idx_ref[g, 0, k] for k in range(_TCROWS)])
    out_ref[...] = (col == tgt[:, None]).astype(jnp.float32)


def _tc_onehot(idx_tc, y_sc):
    return pl.pallas_call(
        _onehot_block,
        grid=(_TCR // _TCROWS,),
        in_specs=[
            pl.BlockSpec(memory_space=pltpu.SMEM),
            pl.BlockSpec(memory_space=pl.ANY),
        ],
        out_specs=pl.BlockSpec((_TCROWS, _N), lambda i: (i, 0)),
        out_shape=jax.ShapeDtypeStruct((_B, _N), jnp.float32),
        input_output_aliases={1: 0},
    )(idx_tc, y_sc)


def kernel(tensor):
    y_sc = _sc_partial(tensor)
    idx_tc = _tc_argmax(tensor)
    return _tc_onehot(idx_tc, y_sc)


# speedup vs baseline: 1.2941x; 1.0093x over previous
import jax
import jax.numpy as jnp
from jax import lax
from jax.experimental import pallas as pl
from jax.experimental.pallas import tpu as pltpu
from jax.experimental.pallas import tpu_sc as plsc

_B = 128
_N = 32768
_L = 16
_NC = 2
_NS = 16
_NW = _NC * _NS
_SCROWS = _NW
_TCR = _B - _SCROWS
_QPR = 8
_Q = _N // _QPR
_TCROWS = 48


def _sc_body(in_hbm, out_hbm, *refs):
    inbufs = refs[:_QPR]
    zbuf = refs[_QPR]
    patch = refs[_QPR + 1]
    in_sems = refs[_QPR + 2 : 2 * _QPR + 2]
    sem_z = refs[2 * _QPR + 2]
    sem_p = refs[2 * _QPR + 3]

    cid = lax.axis_index("c")
    sid = lax.axis_index("s")
    wid = sid * _NC + cid
    row = _TCR + wid

    in_handles = [
        pltpu.async_copy(
            in_hbm.at[row, pl.ds(p * _Q, _Q)], inbufs[p], in_sems[p]
        )
        for p in range(_QPR)
    ]

    zero16 = jnp.zeros((_L,), jnp.float32)

    def zero_body(i, _):
        zbuf[pl.ds(i * _L, _L)] = zero16
        return 0

    lax.fori_loop(0, _Q // _L, zero_body, 0, unroll=8)
    z_handles = [
        pltpu.async_copy(zbuf, out_hbm.at[row, pl.ds(p * _Q, _Q)], sem_z)
        for p in range(_QPR)
    ]

    lane = lax.iota(jnp.int32, _L)
    neg_inf = jnp.full((_L,), -jnp.inf, jnp.float32)

    def make_argmax_body(buf, base):
        def argmax_body(i, carry):
            v0, i0, v1, i1 = carry
            cur = base + i * (2 * _L) + lane
            x0 = buf[pl.ds(i * (2 * _L), _L)]
            x1 = buf[pl.ds(i * (2 * _L) + _L, _L)]
            b0 = x0 > v0
            b1 = x1 > v1
            v0 = jnp.where(b0, x0, v0)
            i0 = jnp.where(b0, cur, i0)
            v1 = jnp.where(b1, x1, v1)
            i1 = jnp.where(b1, cur + _L, i1)
            return v0, i0, v1, i1

        return argmax_body

    carry = (neg_inf, lane, neg_inf, lane + _L)
    for p in range(_QPR):
        in_handles[p].wait()
        carry = lax.fori_loop(
            0, _Q // (2 * _L), make_argmax_body(inbufs[p], p * _Q), carry,
            unroll=4,
        )
    v0, i0, v1, i1 = carry
    take1 = jnp.logical_or(v1 > v0, jnp.logical_and(v1 == v0, i1 < i0))
    best_v = jnp.where(take1, v1, v0)
    best_i = jnp.where(take1, i1, i0)
    for s in (8, 4, 2, 1):
        perm = jnp.bitwise_xor(lane, s)
        ov = best_v.at[perm].get(mode="promise_in_bounds")
        oi = best_i.at[perm].get(mode="promise_in_bounds")
        better = jnp.logical_or(
            ov > best_v, jnp.logical_and(ov == best_v, oi < best_i)
        )
        best_v = jnp.where(better, ov, best_v)
        best_i = jnp.where(better, oi, best_i)
    idx = best_i[0]
    off = jnp.bitwise_and(idx, _L - 1)
    blk = pl.multiple_of(jnp.bitwise_and(idx, -_L), _L)
    patch[:] = jnp.where(lane == off, 1.0, 0.0).astype(jnp.float32)

    for h in z_handles:
        h.wait()
    pltpu.async_copy(
        patch, out_hbm.at[row, pl.ds(blk, _L)], sem_p
    ).wait()


def _sc_partial(tensor):
    mesh = plsc.VectorSubcoreMesh(
        core_axis_name="c", subcore_axis_name="s", num_cores=_NC, num_subcores=_NS
    )
    f = pl.kernel(
        _sc_body,
        out_type=jax.ShapeDtypeStruct((_B, _N), jnp.float32),
        mesh=mesh,
        scratch_types=(
            [pltpu.VMEM((_Q,), jnp.float32) for _ in range(_QPR)]
            + [pltpu.VMEM((_Q,), jnp.float32), pltpu.VMEM((_L,), jnp.float32)]
            + [pltpu.SemaphoreType.DMA for _ in range(_QPR + 2)]
        ),
    )
    return f(tensor)


def _tc_argmax_block(in_ref, out_ref):
    x = in_ref[...]
    col = lax.broadcasted_iota(jnp.int32, (_TCROWS, _N), 1)
    m = jnp.max(x, axis=1, keepdims=True)
    idx = jnp.min(jnp.where(x == m, col, _N), axis=1)
    out_ref[...] = idx.reshape(1, 1, _TCROWS)


def _tc_argmax(tensor):
    nblk = _TCR // _TCROWS
    return pl.pallas_call(
        _tc_argmax_block,
        grid=(nblk,),
        in_specs=[pl.BlockSpec((_TCROWS, _N), lambda i: (i, 0))],
        out_specs=pl.BlockSpec((1, 1, _TCROWS), lambda i: (i, 0, 0)),
        out_shape=jax.ShapeDtypeStruct((nblk, 1, _TCROWS), jnp.int32),
    )(tensor)


def _onehot_block(idx_ref, carry_ref, out_ref):
    del carry_ref
    g = pl.program_id(0)
    col = lax.broadcasted_iota(jnp.int32, (_TCROWS, _N), 1)
    tgt = jnp.stack([idx_ref[g, 0, k] for k in range(_TCROWS)])
    out_ref[...] = (col == tgt[:, None]).astype(jnp.float32)


def _tc_onehot(idx_tc, y_sc):
    return pl.pallas_call(
        _onehot_block,
        grid=(_TCR // _TCROWS,),
        in_specs=[
            pl.BlockSpec(memory_space=pltpu.SMEM),
            pl.BlockSpec(memory_space=pl.ANY),
        ],
        out_specs=pl.BlockSpec((_TCROWS, _N), lambda i: (i, 0)),
        out_shape=jax.ShapeDtypeStruct((_B, _N), jnp.float32),
        input_output_aliases={1: 0},
    )(idx_tc, y_sc)


def kernel(tensor):
    y_sc = _sc_partial(tensor)
    idx_tc = _tc_argmax(tensor)
    return _tc_onehot(idx_tc, y_sc)
